# baseline (speedup 1.0000x reference)
import jax
import jax.numpy as jnp
from jax.experimental import pallas as pl

F = 16
NF = 128
C16 = F * 16
C8 = F * 8
EPS = 1e-12


def _fused_kernel(ht_ref, w1_ref, b1_ref, adj_ref, w3_ref, b3_ref,
                  a16_ref, l1_ref, a8_ref, g_ref, l2_ref,
                  wpre_ref, bpre_ref, bl1_ref, w8a_ref, b8a_ref, gb_ref,
                  w8b_ref, b8b_ref, bl2_ref, out_ref):
    f32 = jnp.float32
    H = ht_ref[...]
    X = jnp.dot(w1_ref[...], H, preferred_element_type=f32) + b1_ref[...]

    A16 = a16_ref[...]
    U = jnp.dot(X, A16, preferred_element_type=f32)
    Xc = X - U
    V = jnp.dot(Xc * Xc, A16, preferred_element_type=f32)
    Tt = jnp.maximum(wpre_ref[...] * (Xc * jax.lax.rsqrt(V + EPS)) + bpre_ref[...], 0.0)

    Y = jnp.dot(Tt, l1_ref[...], preferred_element_type=f32) + bl1_ref[...]

    A8 = a8_ref[...]
    U = jnp.dot(Y, A8, preferred_element_type=f32)
    Yc = Y - U
    V = jnp.dot(Yc * Yc, A8, preferred_element_type=f32)
    Y = jnp.maximum(w8a_ref[...] * (Yc * jax.lax.rsqrt(V + EPS)) + b8a_ref[...], 0.0)

    adj = adj_ref[...]
    G = g_ref[...]
    gb = gb_ref[...]
    Y = jnp.dot(adj, jnp.dot(Y, G, preferred_element_type=f32),
                preferred_element_type=f32) + gb
    Y = jnp.dot(adj, jnp.dot(Y, G, preferred_element_type=f32),
                preferred_element_type=f32) + gb

    U = jnp.dot(Y, A8, preferred_element_type=f32)
    Yc = Y - U
    V = jnp.dot(Yc * Yc, A8, preferred_element_type=f32)
    Tt = jnp.maximum(w8b_ref[...] * (Yc * jax.lax.rsqrt(V + EPS)) + b8b_ref[...], 0.0)

    Z = X + jnp.dot(Tt, l2_ref[...], preferred_element_type=f32) + bl2_ref[...]
    out_ref[...] = jnp.dot(w3_ref[...], Z, preferred_element_type=f32) + b3_ref[...]


def kernel(hidden_states, W1, b1, ln_pre_w, ln_pre_b, lin1_w, lin1_b,
           ln1_w, ln1_b, gcn_w, gcn_b, adjmat, ln2_w, ln2_b,
           lin2_w, lin2_b, W3, b3):
    B, C, T = hidden_states.shape[:3]
    f32 = jnp.float32

    Ht = hidden_states.reshape(NF, C, 16).transpose(1, 0, 2).reshape(C, NF * 16)

    eyeF = jnp.eye(F, dtype=f32)
    A16 = jnp.kron(eyeF, jnp.full((16, 16), 1.0 / 16.0, f32))
    L1 = jnp.kron(eyeF, lin1_w.T)
    A8 = jnp.kron(eyeF, jnp.full((8, 8), 1.0 / 8.0, f32))
    G = jnp.kron(eyeF, gcn_w)
    L2 = jnp.kron(eyeF, lin2_w.T)
    wpre = jnp.tile(ln_pre_w, F)[None, :]
    bpre = jnp.tile(ln_pre_b, F)[None, :]
    bl1 = jnp.tile(lin1_b, F)[None, :]
    w8a = jnp.tile(ln1_w, F)[None, :]
    b8a = jnp.tile(ln1_b, F)[None, :]
    gb = jnp.tile(gcn_b, F)[None, :]
    w8b = jnp.tile(ln2_w, F)[None, :]
    b8b = jnp.tile(ln2_b, F)[None, :]
    bl2 = jnp.tile(lin2_b, F)[None, :]
    b1c = b1[:, None]
    b3c = b3[:, None]

    const = lambda i: (0, 0)
    grid = NF // F
    out = pl.pallas_call(
        _fused_kernel,
        grid=(grid,),
        in_specs=[
            pl.BlockSpec((C, C16), lambda i: (0, i)),
            pl.BlockSpec((431, C), const),
            pl.BlockSpec((431, 1), const),
            pl.BlockSpec((431, 431), const),
            pl.BlockSpec((C, 431), const),
            pl.BlockSpec((C, 1), const),
            pl.BlockSpec((C16, C16), const),
            pl.BlockSpec((C16, C8), const),
            pl.BlockSpec((C8, C8), const),
            pl.BlockSpec((C8, C8), const),
            pl.BlockSpec((C8, C16), const),
            pl.BlockSpec((1, C16), const),
            pl.BlockSpec((1, C16), const),
            pl.BlockSpec((1, C8), const),
            pl.BlockSpec((1, C8), const),
            pl.BlockSpec((1, C8), const),
            pl.BlockSpec((1, C8), const),
            pl.BlockSpec((1, C8), const),
            pl.BlockSpec((1, C8), const),
            pl.BlockSpec((1, C16), const),
        ],
        out_specs=pl.BlockSpec((C, C16), lambda i: (0, i)),
        out_shape=jax.ShapeDtypeStruct((C, NF * 16), f32),
    )(Ht, W1, b1c, adjmat, W3, b3c, A16, L1, A8, G, L2,
      wpre, bpre, bl1, w8a, b8a, gb, w8b, b8b, bl2)

    return out.reshape(C, NF, 16).transpose(1, 0, 2).reshape(B, C, T, 4, 4)
